# baseline (speedup 1.0000x reference)
import functools

import jax
import jax.numpy as jnp
from jax import lax
from jax.experimental import pallas as pl
from jax.experimental.pallas import tpu as pltpu
from jax.experimental.pallas import tpu_sc as plsc

_K = 512
_D = 256
_BLK = 2048
_N = 8192


def _vq_body(x_ref, cb_ref, idx_ref, loss_ref, csq_ref):
    x = x_ref[...]
    cb = cb_ref[...]

    @pl.when(pl.program_id(0) == 0)
    def _():
        csq_ref[...] = jnp.sum(cb * cb, axis=1, keepdims=True).reshape(1, _K)
        loss_ref[0, 0] = 0.0

    cross = jax.lax.dot_general(
        x, cb, (((1,), (1,)), ((), ())), preferred_element_type=jnp.float32
    )
    x_sq = jnp.sum(x * x, axis=1, keepdims=True)
    c_sq = csq_ref[...]
    dist_sq = jnp.maximum(x_sq + c_sq - 2.0 * cross, 0.0)
    dists = jnp.sqrt(dist_sq)
    min_d = jnp.min(dists, axis=1, keepdims=True)
    iota_f = jax.lax.broadcasted_iota(jnp.int32, dists.shape, 1).astype(jnp.float32)
    idx_f = jnp.min(
        jnp.where(dists == min_d, iota_f, float(_K)), axis=1, keepdims=True
    )
    idx_ref[...] = idx_f.astype(jnp.int32).reshape(1, 1, _BLK)
    loss_ref[0, 0] += jnp.sum(min_d * min_d)


_SC_INFO = plsc.get_sparse_core_info()
_NW = _SC_INFO.num_cores * _SC_INFO.num_subcores
_B_PER_W = _N // _NW


@functools.partial(
    pl.kernel,
    mesh=plsc.VectorSubcoreMesh(core_axis_name="c", subcore_axis_name="s"),
    out_type=jax.ShapeDtypeStruct((_N, _D), jnp.float32),
    scratch_types=[
        pltpu.VMEM((_B_PER_W,), jnp.int32),
        pltpu.VMEM((_B_PER_W, _D), jnp.float32),
        pltpu.SemaphoreType.DMA,
    ],
)
def _sc_gather(table_hbm, idx_hbm, out_hbm, idx_v, rows_v, sem):
    wid = lax.axis_index("s") * _SC_INFO.num_cores + lax.axis_index("c")
    base = wid * _B_PER_W
    pltpu.sync_copy(idx_hbm.at[pl.ds(base, _B_PER_W)], idx_v)
    pltpu.async_copy(table_hbm.at[idx_v], rows_v, sem).wait()
    pltpu.sync_copy(rows_v, out_hbm.at[pl.ds(base, _B_PER_W)])


def kernel(x, codebook):
    B, T, D = x.shape
    N = B * T
    xf = x.reshape(N, D)
    grid = N // _BLK
    idx, loss_sum = pl.pallas_call(
        _vq_body,
        grid=(grid,),
        in_specs=[
            pl.BlockSpec((_BLK, D), lambda i: (i, 0)),
            pl.BlockSpec((_K, D), lambda i: (0, 0)),
        ],
        out_specs=[
            pl.BlockSpec((1, 1, _BLK), lambda i: (i, 0, 0)),
            pl.BlockSpec(memory_space=pltpu.SMEM),
        ],
        out_shape=[
            jax.ShapeDtypeStruct((grid, 1, _BLK), jnp.int32),
            jax.ShapeDtypeStruct((1, 1), jnp.float32),
        ],
        scratch_shapes=[pltpu.VMEM((1, _K), jnp.float32)],
    )(xf, codebook)
    indices = idx.reshape(B, T)
    quantized = _sc_gather(codebook, idx.reshape(N)).reshape(B, T, D)
    quantize_loss = (2.0 / N / D) * loss_sum[0, 0]
    return (quantized, indices, quantize_loss)


# speedup vs baseline: 1.7471x; 1.7471x over previous
import jax
import jax.numpy as jnp
from jax.experimental import pallas as pl
from jax.experimental.pallas import tpu as pltpu

_K = 512
_D = 256
_BLK = 2048


def _vq_body(x_ref, cb_ref, quant_ref, idx_ref, loss_ref, csq_ref):
    x = x_ref[...]
    cb = cb_ref[...]

    @pl.when(pl.program_id(0) == 0)
    def _():
        csq_ref[...] = jnp.sum(cb * cb, axis=1, keepdims=True).reshape(1, _K)
        loss_ref[0, 0] = 0.0

    cross = jax.lax.dot_general(
        x, cb, (((1,), (1,)), ((), ())), preferred_element_type=jnp.float32
    )
    x_sq = jnp.sum(x * x, axis=1, keepdims=True)
    c_sq = csq_ref[...]
    dist_sq = jnp.maximum(x_sq + c_sq - 2.0 * cross, 0.0)
    m2 = jnp.min(dist_sq, axis=1, keepdims=True)
    s_min = jnp.sqrt(m2)
    iota_f = jax.lax.broadcasted_iota(jnp.int32, dist_sq.shape, 1).astype(jnp.float32)
    idx_f = jnp.min(
        jnp.where(jnp.sqrt(dist_sq) == s_min, iota_f, float(_K)),
        axis=1, keepdims=True,
    )
    idx_ref[...] = idx_f.astype(jnp.int32).reshape(1, 1, _BLK)
    iota_g = jax.lax.broadcasted_iota(jnp.int32, dist_sq.shape, 1).astype(jnp.float32)
    onehot = (iota_g == idx_f).astype(jnp.bfloat16)
    gathered = jax.lax.dot_general(
        onehot, cb.astype(jnp.bfloat16), (((1,), (0,)), ((), ())),
        preferred_element_type=jnp.float32,
    )
    quant_ref[...] = gathered
    loss_ref[0, 0] += jnp.sum(m2)


def kernel(x, codebook):
    B, T, D = x.shape
    N = B * T
    xf = x.reshape(N, D)
    grid = N // _BLK
    quant, idx, loss_sum = pl.pallas_call(
        _vq_body,
        grid=(grid,),
        in_specs=[
            pl.BlockSpec((_BLK, D), lambda i: (i, 0)),
            pl.BlockSpec((_K, D), lambda i: (0, 0)),
        ],
        out_specs=[
            pl.BlockSpec((_BLK, D), lambda i: (i, 0)),
            pl.BlockSpec((1, 1, _BLK), lambda i: (i, 0, 0)),
            pl.BlockSpec(memory_space=pltpu.SMEM),
        ],
        out_shape=[
            jax.ShapeDtypeStruct((N, D), jnp.float32),
            jax.ShapeDtypeStruct((grid, 1, _BLK), jnp.int32),
            jax.ShapeDtypeStruct((1, 1), jnp.float32),
        ],
        scratch_shapes=[pltpu.VMEM((1, _K), jnp.float32)],
    )(xf, codebook)
    quantized = quant.reshape(B, T, D)
    indices = idx.reshape(B, T)
    quantize_loss = (2.0 / N / D) * loss_sum[0, 0]
    return (quantized, indices, quantize_loss)


# speedup vs baseline: 1.8681x; 1.0692x over previous
import jax
import jax.numpy as jnp
from jax.experimental import pallas as pl
from jax.experimental.pallas import tpu as pltpu

_K = 512
_D = 256
_BLK = 2048


def _vq_body(x_ref, cb_ref, quant_ref, idx_ref, loss_ref, csq_ref):
    x = x_ref[...]
    cb = cb_ref[...]

    @pl.when(pl.program_id(0) == 0)
    def _():
        csq_ref[...] = jnp.sum(cb * cb, axis=1, keepdims=True).reshape(1, _K)
        loss_ref[0, 0] = 0.0

    cross = jax.lax.dot_general(
        x, cb, (((1,), (1,)), ((), ())), preferred_element_type=jnp.float32
    )
    x_sq = jnp.sum(x * x, axis=1, keepdims=True)
    c_sq = csq_ref[...]
    dists = jnp.sqrt(jnp.maximum(x_sq + c_sq - 2.0 * cross, 0.0))
    min_d = jnp.min(dists, axis=1, keepdims=True)
    iota_i = jax.lax.broadcasted_iota(jnp.int32, dists.shape, 1)
    iota_f = iota_i.astype(jnp.float32)
    idx_f = jnp.min(
        jnp.where(dists == min_d, iota_f, float(_K)), axis=1, keepdims=True
    )
    idx_i = idx_f.astype(jnp.int32)
    idx_ref[...] = idx_i.reshape(1, 1, _BLK)
    onehot = (iota_i == idx_i).astype(jnp.bfloat16)
    gathered = jax.lax.dot_general(
        onehot, cb.astype(jnp.bfloat16), (((1,), (0,)), ((), ())),
        preferred_element_type=jnp.float32,
    )
    quant_ref[...] = gathered
    loss_ref[0, 0] += jnp.sum(min_d * min_d)


def kernel(x, codebook):
    B, T, D = x.shape
    N = B * T
    xf = x.reshape(N, D)
    grid = N // _BLK
    quant, idx, loss_sum = pl.pallas_call(
        _vq_body,
        grid=(grid,),
        in_specs=[
            pl.BlockSpec((_BLK, D), lambda i: (i, 0)),
            pl.BlockSpec((_K, D), lambda i: (0, 0)),
        ],
        out_specs=[
            pl.BlockSpec((_BLK, D), lambda i: (i, 0)),
            pl.BlockSpec((1, 1, _BLK), lambda i: (i, 0, 0)),
            pl.BlockSpec(memory_space=pltpu.SMEM),
        ],
        out_shape=[
            jax.ShapeDtypeStruct((N, D), jnp.float32),
            jax.ShapeDtypeStruct((grid, 1, _BLK), jnp.int32),
            jax.ShapeDtypeStruct((1, 1), jnp.float32),
        ],
        scratch_shapes=[pltpu.VMEM((1, _K), jnp.float32)],
    )(xf, codebook)
    quantized = quant.reshape(B, T, D)
    indices = idx.reshape(B, T)
    quantize_loss = (2.0 / N / D) * loss_sum[0, 0]
    return (quantized, indices, quantize_loss)
